# baseline (speedup 1.0000x reference)
import functools

import jax
import jax.numpy as jnp
from jax import lax
from jax.experimental import pallas as pl
from jax.experimental.pallas import tpu as pltpu
from jax.experimental.pallas import tpu_sc as plsc

_IN_DIM = 1000000
_OUT_DIM = 64
_BATCH = 16384

_NC = 2
_NS = 16
_NW = _NC * _NS
_BPW = _BATCH // _NW
_CHUNK = 128
_NCHUNK = _BPW // _CHUNK


@functools.partial(
    pl.kernel,
    mesh=plsc.VectorSubcoreMesh(core_axis_name="c", subcore_axis_name="s"),
    out_type=jax.ShapeDtypeStruct((_NW, _NCHUNK, _CHUNK, _OUT_DIM), jnp.float32),
    scratch_types=[
        pltpu.VMEM((_NCHUNK, _CHUNK), jnp.int32),
        pltpu.VMEM((_NCHUNK, _CHUNK, _OUT_DIM), jnp.float32),
        pltpu.SemaphoreType.DMA,
    ],
    compiler_params=pltpu.CompilerParams(use_tc_tiling_on_sc=False),
)
def _sc_gather(idx_hbm, table_hbm, out_hbm, idx_v, rows_v, sem):
    wid = lax.axis_index("s") * _NC + lax.axis_index("c")
    pltpu.sync_copy(idx_hbm.at[pl.ds(wid * _NCHUNK, _NCHUNK)], idx_v)
    for j in range(_NCHUNK):
        pltpu.async_copy(table_hbm.at[idx_v.at[j]], rows_v.at[j], sem)
    for _ in range(_NCHUNK):
        pltpu.make_async_copy(table_hbm.at[idx_v.at[0]], rows_v.at[0], sem).wait()
    pltpu.sync_copy(rows_v, out_hbm.at[wid])


def kernel(x, ker):
    idx = jnp.reshape(x, (_NW * _NCHUNK, _CHUNK)).astype(jnp.int32)
    out = _sc_gather(idx, ker)
    return jnp.reshape(out, (_BATCH, _OUT_DIM))


# speedup vs baseline: 2.1163x; 2.1163x over previous
import functools

import jax
import jax.numpy as jnp
from jax import lax
from jax.experimental import pallas as pl
from jax.experimental.pallas import tpu as pltpu
from jax.experimental.pallas import tpu_sc as plsc

_IN_DIM = 1000000
_OUT_DIM = 64
_BATCH = 16384
_TROW = 8

_NC = 2
_NS = 16
_NW = _NC * _NS
_BPW = _BATCH // _NW
_CHUNK = 16
_NCHUNK = _BPW // _CHUNK
_NBUF = 2
_L = 16


def _extract(vec, lane):
    onehot = lax.iota(jnp.int32, _L) == lane
    return jnp.sum(jnp.where(onehot, vec, 0))


@functools.partial(
    pl.kernel,
    mesh=plsc.VectorSubcoreMesh(core_axis_name="c", subcore_axis_name="s"),
    out_type=jax.ShapeDtypeStruct((_NW, _BPW, _OUT_DIM), jnp.float32),
    scratch_types=[
        pltpu.VMEM((_NCHUNK, _CHUNK), jnp.int32),
        pltpu.VMEM((_NBUF, _CHUNK, _TROW, _OUT_DIM), jnp.float32),
        pltpu.VMEM((_BPW, _OUT_DIM), jnp.float32),
        pltpu.SemaphoreType.DMA,
        pltpu.SemaphoreType.DMA,
    ],
    compiler_params=pltpu.CompilerParams(use_tc_tiling_on_sc=True,
                                         needs_layout_passes=False),
)
def _sc_gather(idx_hbm, table_hbm, out_hbm, idx_v, slab_v, rows_v, sem0,
               sem1):
    sems = [sem0, sem1]
    wid = lax.axis_index("s") * _NC + lax.axis_index("c")
    pltpu.sync_copy(idx_hbm.at[wid], idx_v)

    def fire(c):
        b = c % _NBUF

        def body(i, _):
            vec = idx_v[c, pl.ds(lax.shift_left(lax.shift_right_logical(i, 4), 4), _L)]
            v = _extract(vec, lax.bitwise_and(i, _L - 1))
            t = lax.shift_right_logical(v, 3)
            pltpu.async_copy(table_hbm.at[t], slab_v.at[b, i], sems[b])
            return 0

        lax.fori_loop(0, _CHUNK, body, 0)

    def wait(c):
        b = c % _NBUF

        def body(i, _):
            pltpu.make_async_copy(table_hbm.at[0], slab_v.at[b, 0],
                                  sems[b]).wait()
            return 0

        lax.fori_loop(0, _CHUNK, body, 0)

    def extract(c):
        b = c % _NBUF

        def body(i, _):
            vec = idx_v[c, pl.ds(lax.shift_left(lax.shift_right_logical(i, 4), 4), _L)]
            v = _extract(vec, lax.bitwise_and(i, _L - 1))
            r = lax.bitwise_and(v, _TROW - 1)
            row = c * _CHUNK + i
            for q in range(_OUT_DIM // _L):
                rows_v[row, pl.ds(q * _L, _L)] = slab_v[b, i, r,
                                                        pl.ds(q * _L, _L)]
            return 0

        lax.fori_loop(0, _CHUNK, body, 0)

    fire(0)
    for c in range(_NCHUNK):
        wait(c)
        if c + 1 < _NCHUNK:
            fire(c + 1)
        extract(c)
    pltpu.sync_copy(rows_v, out_hbm.at[wid])


def kernel(x, ker):
    idx = jnp.reshape(x, (_NW, _NCHUNK, _CHUNK)).astype(jnp.int32)
    table = jnp.reshape(ker, (_IN_DIM // _TROW, _TROW, _OUT_DIM))
    out = _sc_gather(idx, table)
    return jnp.reshape(out, (_BATCH, _OUT_DIM))
